# baseline (speedup 1.0000x reference)
import functools

import jax
import jax.numpy as jnp
from jax import lax
from jax.experimental import pallas as pl
from jax.experimental.pallas import tpu as pltpu
from jax.experimental.pallas import tpu_sc as plsc

N = 320000
D = 128
S = 10000


_BR = 4000
_NBUF = 8


def _matvec_body(b_ref, w_ref, p1_hbm, q_ref, buf, sems):
    g = pl.program_id(0)
    ng = pl.num_programs(0)

    @pl.when(g == 0)
    def _():
        for k in range(_NBUF):
            pltpu.async_copy(
                p1_hbm.at[pl.ds(k * _BR, _BR), :], buf.at[k], sems.at[k]
            )

    slot = lax.rem(g, _NBUF)
    pltpu.make_async_copy(
        p1_hbm.at[pl.ds(0, _BR), :], buf.at[slot], sems.at[slot]
    ).wait()
    q_ref[...] = (
        jnp.dot(buf[slot], w_ref[...], preferred_element_type=jnp.float32)
        + b_ref[0]
    )

    @pl.when(g + _NBUF < ng)
    def _():
        pltpu.async_copy(
            p1_hbm.at[pl.ds((g + _NBUF) * _BR, _BR), :],
            buf.at[slot],
            sems.at[slot],
        )


def _tc_matvec(p1, W, b):
    grid = (N // _BR,)
    return pl.pallas_call(
        _matvec_body,
        grid=grid,
        in_specs=[
            pl.BlockSpec(memory_space=pltpu.SMEM),
            pl.BlockSpec((D, 1), lambda i: (0, 0)),
            pl.BlockSpec(memory_space=pltpu.MemorySpace.HBM),
        ],
        out_specs=pl.BlockSpec((_BR, 1), lambda i: (i, 0)),
        out_shape=jax.ShapeDtypeStruct((N, 1), jnp.float32),
        scratch_shapes=[
            pltpu.VMEM((_NBUF, _BR, D), jnp.float32),
            pltpu.SemaphoreType.DMA((_NBUF,)),
        ],
    )(b, W, p1)


_NT = 16
_CHUNK = N // _NT
_SPAD = 10240
_SLC = _SPAD // _NT
_L = 16

_sc_mesh = plsc.VectorSubcoreMesh(
    core_axis_name="c", subcore_axis_name="s", num_cores=1
)


@functools.partial(
    pl.kernel,
    out_type=jax.ShapeDtypeStruct((N,), jnp.float32),
    mesh=_sc_mesh,
    scratch_types=[
        pltpu.VMEM((_CHUNK,), jnp.int32),
        pltpu.VMEM((_CHUNK,), jnp.float32),
        pltpu.VMEM((_SPAD,), jnp.float32),
        pltpu.VMEM((_SPAD,), jnp.float32),
        pltpu.VMEM((_NT, _SLC), jnp.float32),
        pltpu.VMEM((_SPAD,), jnp.float32),
        pltpu.VMEM_SHARED((_NT, _SPAD), jnp.float32),
        pltpu.VMEM_SHARED((_NT, _SPAD), jnp.float32),
        pltpu.VMEM_SHARED((_SPAD,), jnp.float32),
        pltpu.SemaphoreType.DMA,
        pltpu.SemaphoreType.DMA,
    ],
    compiler_params=pltpu.CompilerParams(needs_layout_passes=False),
)
def _sc_segment_fix(ids_hbm, q_hbm, out_hbm,
                    ids_v, q_v, lsum, lcnt, red, pch, ssum, scnt, spch,
                    sem_a, sem_b):
    sid = lax.axis_index("s")
    base = sid * _CHUNK
    cp_ids = pltpu.async_copy(ids_hbm.at[pl.ds(base, _CHUNK)], ids_v, sem_a)
    cp_q = pltpu.async_copy(q_hbm.at[pl.ds(base, _CHUNK)], q_v, sem_b)

    zero16 = jnp.zeros((_L,), jnp.float32)
    one16 = jnp.ones((_L,), jnp.float32)

    def zbody(i, _):
        lsum[pl.ds(i * _L, _L)] = zero16
        lcnt[pl.ds(i * _L, _L)] = zero16
        return 0

    lax.fori_loop(0, _SPAD // _L, zbody, 0, unroll=4)

    cp_ids.wait()
    cp_q.wait()

    def sbody(i, _):
        idx = ids_v[pl.ds(i * _L, _L)]
        vals = q_v[pl.ds(i * _L, _L)]
        plsc.addupdate_scatter(lsum, [idx], vals)
        plsc.addupdate_scatter(lcnt, [idx], one16)
        return 0

    lax.fori_loop(0, _CHUNK // _L, sbody, 0, unroll=4)

    pltpu.sync_copy(lsum, ssum.at[sid])
    pltpu.sync_copy(lcnt, scnt.at[sid])
    plsc.subcore_barrier()

    seg0 = sid * _SLC

    def _reduce_banks(bank, dst):
        cps = [
            pltpu.async_copy(bank.at[r, pl.ds(seg0, _SLC)], red.at[r], sem_a)
            for r in range(_NT)
        ]
        for cp in cps:
            cp.wait()

        def rbody(j, _):
            a = red[0, pl.ds(j * _L, _L)]
            for r in range(1, _NT):
                a = a + red[r, pl.ds(j * _L, _L)]
            dst[pl.ds(j * _L, _L)] = a
            return 0

        lax.fori_loop(0, _SLC // _L, rbody, 0, unroll=2)

    _reduce_banks(ssum, lsum)
    _reduce_banks(scnt, lcnt)

    def dbody(j, _):
        lsum[pl.ds(j * _L, _L)] = (
            lsum[pl.ds(j * _L, _L)] / lcnt[pl.ds(j * _L, _L)]
        )
        return 0

    lax.fori_loop(0, _SLC // _L, dbody, 0, unroll=4)

    pltpu.sync_copy(lsum.at[pl.ds(0, _SLC)], spch.at[pl.ds(seg0, _SLC)])
    plsc.subcore_barrier()
    pltpu.sync_copy(spch, pch)

    def gbody(i, _):
        idx = ids_v[pl.ds(i * _L, _L)]
        corr = plsc.load_gather(pch, [idx])
        q_v[pl.ds(i * _L, _L)] = q_v[pl.ds(i * _L, _L)] - corr
        return 0

    lax.fori_loop(0, _CHUNK // _L, gbody, 0, unroll=4)

    pltpu.sync_copy(q_v, out_hbm.at[pl.ds(base, _CHUNK)])


def kernel(atom_batch, p1, W, b):
    ids32 = atom_batch.astype(jnp.int32)
    q = _tc_matvec(p1, W, b).reshape(-1)
    return q.reshape(-1, 1)
    out = _sc_segment_fix(ids32, q)
    return out.reshape(-1, 1)


# speedup vs baseline: 1.5137x; 1.5137x over previous
import functools

import jax
import jax.numpy as jnp
from jax import lax
from jax.experimental import pallas as pl
from jax.experimental.pallas import tpu as pltpu
from jax.experimental.pallas import tpu_sc as plsc

N = 320000
D = 128
S = 10000

_NT = 16
_NW = 32
_CH = N // _NW
_PR = 400
_NP = _CH // _PR
_SPAD = 10240
_SLC = _SPAD // _NT
_L = 16

_mesh = plsc.VectorSubcoreMesh(core_axis_name="c", subcore_axis_name="s")
_params = pltpu.CompilerParams(needs_layout_passes=False)


@functools.partial(
    pl.kernel,
    out_type=[
        jax.ShapeDtypeStruct((N,), jnp.float32),
        jax.ShapeDtypeStruct((_NW, _SPAD), jnp.float32),
        jax.ShapeDtypeStruct((_NW, _SPAD), jnp.float32),
    ],
    mesh=_mesh,
    scratch_types=[
        pltpu.VMEM((2, _PR, D), jnp.float32),
        pltpu.VMEM((2 * _PR,), jnp.int32),
        pltpu.VMEM((2 * _PR,), jnp.float32),
        pltpu.VMEM((D,), jnp.float32),
        pltpu.VMEM((_SPAD,), jnp.float32),
        pltpu.VMEM((_SPAD,), jnp.float32),
        pltpu.SemaphoreType.DMA,
        pltpu.SemaphoreType.DMA,
        pltpu.SemaphoreType.DMA,
    ],
    compiler_params=_params,
)
def _sc_mv_scatter(ids_hbm, p1_hbm, w_hbm, q_hbm, sums_hbm, cnts_hbm,
                   pbuf, idbuf, qbuf, w_v, lsum, lcnt, sem_p, sem_i, sem_q):
    cid = lax.axis_index("c")
    sid = lax.axis_index("s")
    wid = sid * 2 + cid
    row0 = wid * _CH

    pltpu.async_copy(w_hbm, w_v, sem_q)
    pltpu.async_copy(p1_hbm.at[pl.ds(row0, _PR), :], pbuf.at[0], sem_p)
    pltpu.async_copy(
        ids_hbm.at[pl.ds(row0, _PR)], idbuf.at[pl.ds(0, _PR)], sem_i
    )

    zero16 = jnp.zeros((_L,), jnp.float32)
    one16 = jnp.ones((_L,), jnp.float32)

    def zbody(i, _):
        lsum[pl.ds(i * _L, _L)] = zero16
        lcnt[pl.ds(i * _L, _L)] = zero16
        return 0

    lax.fori_loop(0, _SPAD // _L, zbody, 0, unroll=4)

    pltpu.make_async_copy(w_hbm, w_v, sem_q).wait()
    ws = [w_v[pl.ds(j * _L, _L)] for j in range(D // _L)]
    lane_iota = lax.iota(jnp.int32, _L)

    def piece(i, _):
        cur = lax.rem(i, 2)
        nxt = lax.rem(i + 1, 2)

        @pl.when(i + 1 < _NP)
        def _():
            pltpu.async_copy(
                p1_hbm.at[pl.ds(row0 + (i + 1) * _PR, _PR), :],
                pbuf.at[nxt], sem_p,
            )
            pltpu.async_copy(
                ids_hbm.at[pl.ds(row0 + (i + 1) * _PR, _PR)],
                idbuf.at[pl.ds(nxt * _PR, _PR)], sem_i,
            )

        pltpu.make_async_copy(
            p1_hbm.at[pl.ds(0, _PR), :], pbuf.at[cur], sem_p
        ).wait()
        pltpu.make_async_copy(
            ids_hbm.at[pl.ds(0, _PR)], idbuf.at[pl.ds(0, _PR)], sem_i
        ).wait()

        @pl.when(i >= 2)
        def _():
            pltpu.make_async_copy(
                qbuf.at[pl.ds(0, _PR)], q_hbm.at[pl.ds(0, _PR)], sem_q
            ).wait()

        def rowg(g, _):
            v = zero16
            for l in range(_L):
                r = g * _L + l
                t = pbuf[cur, r, pl.ds(0, _L)] * ws[0]
                for j in range(1, D // _L):
                    t = t + pbuf[cur, r, pl.ds(j * _L, _L)] * ws[j]
                v = jnp.where(lane_iota == l, jnp.sum(t), v)
            qbuf[pl.ds(cur * _PR + g * _L, _L)] = v
            return 0

        lax.fori_loop(0, _PR // _L, rowg, 0)

        def scb(k, _):
            idx = idbuf[pl.ds(cur * _PR + k * _L, _L)]
            vals = qbuf[pl.ds(cur * _PR + k * _L, _L)]
            plsc.addupdate_scatter(lsum, [idx], vals)
            plsc.addupdate_scatter(lcnt, [idx], one16)
            return 0

        lax.fori_loop(0, _PR // _L, scb, 0)

        pltpu.async_copy(
            qbuf.at[pl.ds(cur * _PR, _PR)],
            q_hbm.at[pl.ds(row0 + i * _PR, _PR)], sem_q,
        )
        return 0

    lax.fori_loop(0, _NP, piece, 0)

    pltpu.make_async_copy(
        qbuf.at[pl.ds(0, _PR)], q_hbm.at[pl.ds(0, _PR)], sem_q
    ).wait()
    pltpu.make_async_copy(
        qbuf.at[pl.ds(0, _PR)], q_hbm.at[pl.ds(0, _PR)], sem_q
    ).wait()

    pltpu.sync_copy(lsum, sums_hbm.at[wid])
    pltpu.sync_copy(lcnt, cnts_hbm.at[wid])


@functools.partial(
    pl.kernel,
    out_type=jax.ShapeDtypeStruct((N,), jnp.float32),
    mesh=_mesh,
    scratch_types=[
        pltpu.VMEM((_CH,), jnp.int32),
        pltpu.VMEM((_CH,), jnp.float32),
        pltpu.VMEM((_SPAD,), jnp.float32),
        pltpu.VMEM((_NW, _SLC), jnp.float32),
        pltpu.VMEM((_NW, _SLC), jnp.float32),
        pltpu.VMEM_SHARED((_SPAD,), jnp.float32),
        pltpu.SemaphoreType.DMA,
        pltpu.SemaphoreType.DMA,
    ],
    compiler_params=_params,
)
def _sc_finish(ids_hbm, q_hbm, sums_hbm, cnts_hbm, out_hbm,
               ids_v, q_v, pch, pblk, cblk, spch, sem_a, sem_b):
    cid = lax.axis_index("c")
    sid = lax.axis_index("s")
    wid = sid * 2 + cid
    base = wid * _CH
    seg0 = sid * _SLC

    cp_i = pltpu.async_copy(ids_hbm.at[pl.ds(base, _CH)], ids_v, sem_a)
    cp_q = pltpu.async_copy(q_hbm.at[pl.ds(base, _CH)], q_v, sem_a)
    cp_s = pltpu.async_copy(sums_hbm.at[:, pl.ds(seg0, _SLC)], pblk, sem_b)
    cp_c = pltpu.async_copy(cnts_hbm.at[:, pl.ds(seg0, _SLC)], cblk, sem_b)
    cp_s.wait()
    cp_c.wait()

    def mbody(j, _):
        s = pblk[0, pl.ds(j * _L, _L)]
        c = cblk[0, pl.ds(j * _L, _L)]
        for r in range(1, _NW):
            s = s + pblk[r, pl.ds(j * _L, _L)]
            c = c + cblk[r, pl.ds(j * _L, _L)]
        pblk[0, pl.ds(j * _L, _L)] = s / c
        return 0

    lax.fori_loop(0, _SLC // _L, mbody, 0, unroll=2)

    pltpu.sync_copy(pblk.at[0], spch.at[pl.ds(seg0, _SLC)])
    plsc.subcore_barrier()
    pltpu.sync_copy(spch, pch)

    cp_i.wait()
    cp_q.wait()

    def gbody(i, _):
        idx = ids_v[pl.ds(i * _L, _L)]
        corr = plsc.load_gather(pch, [idx])
        q_v[pl.ds(i * _L, _L)] = q_v[pl.ds(i * _L, _L)] - corr
        return 0

    lax.fori_loop(0, _CH // _L, gbody, 0, unroll=4)

    pltpu.sync_copy(q_v, out_hbm.at[pl.ds(base, _CH)])


def kernel(atom_batch, p1, W, b):
    ids32 = atom_batch.astype(jnp.int32)
    q, sums, cnts = _sc_mv_scatter(ids32, p1, W.reshape(-1))
    out = _sc_finish(ids32, q, sums, cnts)
    return out.reshape(-1, 1)
